# baseline (speedup 1.0000x reference)
import jax
import jax.numpy as jnp
from jax.experimental import pallas as pl
from jax.experimental.pallas import tpu as pltpu

_B, _S, _H, _I, _E, _R, _K = 1, 2048, 2048, 8192, 4, 256, 2
_ALPHA = 16.0
_ER = _E * _R
_TI = 512
_TS = 1024

_DNT = (((1,), (1,)), ((), ()))


def _moe_body(x_ref, w1_ref, wr_ref, w2_ref, out_ref, aw_ref):
    ii = pl.program_id(1)

    @pl.when(ii == 0)
    def _prologue():
        xb = x_ref[...]
        logits = jnp.dot(xb, wr_ref[...], preferred_element_type=jnp.float32)
        m = jnp.max(logits, axis=1, keepdims=True)
        p = jnp.exp(logits - m)
        w = p / jnp.sum(p, axis=1, keepdims=True)
        col = jax.lax.broadcasted_iota(jnp.int32, (_TS, _E), 1)
        rank = jnp.zeros((_TS, _E), jnp.int32)
        for ep in range(_E):
            wep = w[:, ep:ep + 1]
            beats = (wep > w) | ((wep == w) & (ep < col))
            rank += beats.astype(jnp.int32)
        wsel = jnp.where(rank < _K, w, 0.0)
        c = wsel / (jnp.sum(wsel, axis=1, keepdims=True) + 1e-6) * (_ALPHA / _R)
        xb16 = xb.astype(jnp.bfloat16)
        for e in range(_E):
            sl = slice(e * _R, (e + 1) * _R)
            h = jax.lax.dot_general(xb16, w1_ref[sl, :], _DNT,
                                    preferred_element_type=jnp.float32)
            a = 0.5 * h * (1.0 + jax.lax.erf(h * 0.7071067811865476))
            aw_ref[:, sl] = (a * c[:, e:e + 1]).astype(jnp.bfloat16)

    acc = jax.lax.dot_general(
        aw_ref[:, 0:_R], w2_ref[0].astype(jnp.bfloat16), _DNT,
        preferred_element_type=jnp.float32)
    for e in range(1, _E):
        acc += jax.lax.dot_general(
            aw_ref[:, e * _R:(e + 1) * _R], w2_ref[e].astype(jnp.bfloat16),
            _DNT, preferred_element_type=jnp.float32)
    out_ref[...] = acc


def kernel(x, W1, W2, Wr):
    xs = x.reshape(_S, _H)
    w1r = W1.reshape(_ER, _H).astype(jnp.bfloat16)
    wrt = Wr.T
    out = pl.pallas_call(
        _moe_body,
        grid=(_S // _TS, _I // _TI),
        in_specs=[
            pl.BlockSpec((_TS, _H), lambda si, ii: (si, 0)),
            pl.BlockSpec((_ER, _H), lambda si, ii: (0, 0)),
            pl.BlockSpec((_H, _E), lambda si, ii: (0, 0)),
            pl.BlockSpec((_E, _TI, _R), lambda si, ii: (0, ii, 0)),
        ],
        out_specs=pl.BlockSpec((_TS, _TI), lambda si, ii: (si, ii)),
        out_shape=jax.ShapeDtypeStruct((_S, _I), jnp.float32),
        scratch_shapes=[pltpu.VMEM((_TS, _ER), jnp.bfloat16)],
    )(xs, w1r, wrt, W2)
    return out.reshape(_B, _S, _I)


# speedup vs baseline: 1.2364x; 1.2364x over previous
import jax
import jax.numpy as jnp
from jax.experimental import pallas as pl
from jax.experimental.pallas import tpu as pltpu

_B, _S, _H, _I, _E, _R, _K = 1, 2048, 2048, 8192, 4, 256, 2
_ALPHA = 16.0
_ER = _E * _R
_TI = 512

_DNT = (((1,), (1,)), ((), ()))


def _moe_body(x_ref, w1_ref, wr_ref, w2_ref, out_ref, aw_ref):
    i = pl.program_id(0)

    @pl.when(i == 0)
    def _prologue():
        xb = x_ref[...]
        logits = jnp.dot(xb, wr_ref[...], preferred_element_type=jnp.float32)
        m = jnp.max(logits, axis=1, keepdims=True)
        p = jnp.exp(logits - m)
        w = p / jnp.sum(p, axis=1, keepdims=True)
        col = jax.lax.broadcasted_iota(jnp.int32, (_S, _E), 1)
        rank = jnp.zeros((_S, _E), jnp.int32)
        for ep in range(_E):
            wep = w[:, ep:ep + 1]
            beats = (wep > w) | ((wep == w) & (ep < col))
            rank += beats.astype(jnp.int32)
        wsel = jnp.where(rank < _K, w, 0.0)
        c = wsel / (jnp.sum(wsel, axis=1, keepdims=True) + 1e-6) * (_ALPHA / _R)
        xb16 = xb.astype(jnp.bfloat16)
        for e in range(_E):
            sl = slice(e * _R, (e + 1) * _R)
            h = jax.lax.dot_general(xb16, w1_ref[sl, :], _DNT,
                                    preferred_element_type=jnp.float32)
            a = 0.5 * h * (1.0 + jax.lax.erf(h * 0.7071067811865476))
            aw_ref[:, sl] = (a * c[:, e:e + 1]).astype(jnp.bfloat16)

    acc = jax.lax.dot_general(
        aw_ref[:, 0:_R], w2_ref[0].astype(jnp.bfloat16), _DNT,
        preferred_element_type=jnp.float32)
    for e in range(1, _E):
        acc += jax.lax.dot_general(
            aw_ref[:, e * _R:(e + 1) * _R], w2_ref[e].astype(jnp.bfloat16),
            _DNT, preferred_element_type=jnp.float32)
    out_ref[...] = acc


def kernel(x, W1, W2, Wr):
    xs = x.reshape(_S, _H)
    w1r = W1.reshape(_ER, _H).astype(jnp.bfloat16)
    wrt = Wr.T
    out = pl.pallas_call(
        _moe_body,
        grid=(_I // _TI,),
        in_specs=[
            pl.BlockSpec((_S, _H), lambda i: (0, 0)),
            pl.BlockSpec((_ER, _H), lambda i: (0, 0)),
            pl.BlockSpec((_H, _E), lambda i: (0, 0)),
            pl.BlockSpec((_E, _TI, _R), lambda i: (0, i, 0)),
        ],
        out_specs=pl.BlockSpec((_S, _TI), lambda i: (0, i)),
        out_shape=jax.ShapeDtypeStruct((_S, _I), jnp.float32),
        scratch_shapes=[pltpu.VMEM((_S, _ER), jnp.bfloat16)],
    )(xs, w1r, wrt, W2)
    return out.reshape(_B, _S, _I)
